# Initial kernel scaffold; baseline (speedup 1.0000x reference)
import jax
import jax.numpy as jnp
from jax.experimental import pallas as pl


def kernel(x, router_weights, gate_w, up_w, down_w, w_down, w_up, experts_w12, experts_w3):
    raise NotImplementedError("write your pallas kernel here")


# speedup vs baseline: 1.8955x; 1.8955x over previous
import functools

import jax
import jax.numpy as jnp
from jax import lax
from jax.experimental import pallas as pl
from jax.experimental.pallas import tpu as pltpu
from jax.experimental.pallas import tpu_sc as plsc

D_MODEL = 1024
D_FF = 4096
E = 64
K = 2
BOTTLE = 256
ROUTED_FF = 3072
N = 2048
ROUTE_SCALE = float(2.0 ** 0.5)
MOE_SCALE = 0.5
MAX_LOAD = (N * K // E // 8 + 6) * 8
NUM_SLOTS = E * MAX_LOAD
PAD_ROWS = NUM_SLOTS + MAX_LOAD

NW = 32
CHUNK = (N * K) // NW

_BF = jnp.bfloat16
_F32 = jnp.float32


def _dot(a, b, dims):
    return lax.dot_general(a.astype(_BF), b.astype(_BF), (dims, ((), ())),
                           preferred_element_type=_F32)


def _router_body(h_ref, rw_ref, wd_ref, dph_ref, slots_ref, gmap_ref, geff_ref):
    h = h_ref[...]
    logits = _dot(h, rw_ref[...], ((1,), (1,)))
    scores = jax.nn.sigmoid(logits)
    col = lax.broadcasted_iota(jnp.int32, (N, E), 1)
    m1 = jnp.max(scores, axis=1, keepdims=True)
    i1 = jnp.min(jnp.where(scores == m1, col, E), axis=1, keepdims=True)
    l1 = jnp.sum(jnp.where(col == i1, logits, 0.0), axis=1, keepdims=True)
    scores2 = jnp.where(col == i1, -1.0, scores)
    m2 = jnp.max(scores2, axis=1, keepdims=True)
    i2 = jnp.min(jnp.where(scores2 == m2, col, E), axis=1, keepdims=True)
    l2 = jnp.sum(jnp.where(col == i2, logits, 0.0), axis=1, keepdims=True)
    g1 = jax.nn.sigmoid(l1 - l2)
    g2 = jax.nn.sigmoid(l2 - l1)

    oh1 = (col == i1).astype(_F32)
    oh2 = (col == i2).astype(_F32)
    oht = oh1 + oh2
    r = lax.broadcasted_iota(jnp.int32, (N, N), 0)
    c = lax.broadcasted_iota(jnp.int32, (N, N), 1)
    ltri = (c < r).astype(_F32)
    cnt = _dot(ltri, oht, ((1,), (0,)))
    p1 = jnp.sum(cnt * oh1, axis=1, keepdims=True).astype(jnp.int32)
    p2 = jnp.sum(cnt * oh2, axis=1, keepdims=True).astype(jnp.int32)

    keep1 = p1 < MAX_LOAD
    keep2 = p2 < MAX_LOAD
    slot1 = i1 * MAX_LOAD + p1
    slot2 = i2 * MAX_LOAD + p2
    slots_ref[...] = jnp.concatenate(
        [jnp.where(keep1, slot1, NUM_SLOTS), jnp.where(keep2, slot2, NUM_SLOTS)],
        axis=1)
    gmap_ref[...] = jnp.concatenate(
        [jnp.where(keep1, slot1, 0), jnp.where(keep2, slot2, 0)], axis=1)
    geff_ref[...] = jnp.concatenate(
        [jnp.where(keep1, g1, 0.0), jnp.where(keep2, g2, 0.0)], axis=1)

    dph_ref[...] = _dot(h, wd_ref[...], ((1,), (1,)))


def _router_call(h, router_weights, w_down):
    return pl.pallas_call(
        _router_body,
        out_shape=(
            jax.ShapeDtypeStruct((N, BOTTLE), _F32),
            jax.ShapeDtypeStruct((N, K), jnp.int32),
            jax.ShapeDtypeStruct((N, K), jnp.int32),
            jax.ShapeDtypeStruct((N, K), _F32),
        ),
    )(h, router_weights, w_down)


def _dispatch_call(dph, slots_km):
    mesh = plsc.VectorSubcoreMesh(core_axis_name="c", subcore_axis_name="s")

    @functools.partial(
        pl.kernel,
        out_type=jax.ShapeDtypeStruct((PAD_ROWS, BOTTLE), _F32),
        mesh=mesh,
        scratch_types=[
            pltpu.VMEM((CHUNK,), jnp.int32),
            pltpu.VMEM((CHUNK, BOTTLE), _F32),
            pltpu.SemaphoreType.DMA,
        ],
    )
    def disp(dph_hbm, slots_hbm, pad_hbm, idx_v, rows_v, sem):
        wid = lax.axis_index("s") * 2 + lax.axis_index("c")
        pltpu.sync_copy(slots_hbm.at[pl.ds(wid * CHUNK, CHUNK)], idx_v)
        tok0 = (wid % 16) * CHUNK
        pltpu.sync_copy(dph_hbm.at[pl.ds(tok0, CHUNK)], rows_v)
        pltpu.async_copy(rows_v, pad_hbm.at[idx_v], sem).wait()

    return disp(dph, slots_km)


def _experts_body(pad_ref, w12_ref, w3_ref, out_ref):
    x = pad_ref[...]
    h12 = _dot(x, w12_ref[0], ((1,), (0,)))
    hg = h12[:, :ROUTED_FF]
    hu = h12[:, ROUTED_FF:]
    act = hg * jax.nn.sigmoid(hg) * hu
    out_ref[...] = _dot(act, w3_ref[0], ((1,), (0,)))


def _experts_call(pad, experts_w12, experts_w3):
    return pl.pallas_call(
        _experts_body,
        grid=(E,),
        in_specs=[
            pl.BlockSpec((MAX_LOAD, BOTTLE), lambda e: (e, 0)),
            pl.BlockSpec((1, BOTTLE, 2 * ROUTED_FF), lambda e: (e, 0, 0)),
            pl.BlockSpec((1, ROUTED_FF, BOTTLE), lambda e: (e, 0, 0)),
        ],
        out_specs=pl.BlockSpec((MAX_LOAD, BOTTLE), lambda e: (e, 0)),
        out_shape=jax.ShapeDtypeStruct((NUM_SLOTS, BOTTLE), _F32),
        compiler_params=pltpu.CompilerParams(
            dimension_semantics=("parallel",)),
    )(pad, experts_w12, experts_w3)


def _combine_call(pout, gmap_km):
    mesh = plsc.VectorSubcoreMesh(core_axis_name="c", subcore_axis_name="s")

    @functools.partial(
        pl.kernel,
        out_type=jax.ShapeDtypeStruct((N * K, BOTTLE), _F32),
        mesh=mesh,
        scratch_types=[
            pltpu.VMEM((CHUNK,), jnp.int32),
            pltpu.VMEM((CHUNK, BOTTLE), _F32),
            pltpu.SemaphoreType.DMA,
        ],
    )
    def comb(pout_hbm, gmap_hbm, out_hbm, idx_v, rows_v, sem):
        wid = lax.axis_index("s") * 2 + lax.axis_index("c")
        pltpu.sync_copy(gmap_hbm.at[pl.ds(wid * CHUNK, CHUNK)], idx_v)
        pltpu.async_copy(pout_hbm.at[idx_v], rows_v, sem).wait()
        pltpu.sync_copy(rows_v, out_hbm.at[pl.ds(wid * CHUNK, CHUNK)])

    return comb(pout, gmap_km)


FF_BLK = 512
FF_STEPS = D_FF // FF_BLK


def _shared_body(h_ref, gw_ref, uw_ref, dw_ref, out_ref, acc_ref):
    i = pl.program_id(0)
    h = h_ref[...]
    g = _dot(h, gw_ref[...], ((1,), (1,)))
    u = _dot(h, uw_ref[...], ((1,), (1,)))
    a = g * jax.nn.sigmoid(g) * u
    part = _dot(a, dw_ref[...], ((1,), (1,)))

    @pl.when(i == 0)
    def _():
        acc_ref[...] = part

    @pl.when(i > 0)
    def _():
        acc_ref[...] += part

    @pl.when(i == FF_STEPS - 1)
    def _():
        out_ref[...] = acc_ref[...]


def _shared_call(h, gate_w, up_w, down_w):
    return pl.pallas_call(
        _shared_body,
        grid=(FF_STEPS,),
        in_specs=[
            pl.BlockSpec((N, D_MODEL), lambda i: (0, 0)),
            pl.BlockSpec((FF_BLK, D_MODEL), lambda i: (i, 0)),
            pl.BlockSpec((FF_BLK, D_MODEL), lambda i: (i, 0)),
            pl.BlockSpec((D_MODEL, FF_BLK), lambda i: (0, i)),
        ],
        out_specs=pl.BlockSpec((N, D_MODEL), lambda i: (0, 0)),
        out_shape=jax.ShapeDtypeStruct((N, D_MODEL), _F32),
        scratch_shapes=[pltpu.VMEM((N, D_MODEL), _F32)],
        compiler_params=pltpu.CompilerParams(
            dimension_semantics=("arbitrary",)),
    )(h, gate_w, up_w, down_w)


def _final_body(sh_ref, gath_ref, geff_ref, wu_ref, out_ref):
    g0 = geff_ref[:, 0:1]
    g1 = geff_ref[:, 1:2]
    routed_b = g0 * gath_ref[0:N, :] + g1 * gath_ref[N:2 * N, :]
    routed = _dot(routed_b, wu_ref[...], ((1,), (1,)))
    out_ref[...] = (sh_ref[...] + routed * ROUTE_SCALE) * MOE_SCALE


def _final_call(shared, gathered, geff, w_up):
    return pl.pallas_call(
        _final_body,
        out_shape=jax.ShapeDtypeStruct((N, D_MODEL), _F32),
    )(shared, gathered, geff, w_up)


def kernel(x, router_weights, gate_w, up_w, down_w, w_down, w_up,
           experts_w12, experts_w3):
    Bb, Ss, D = x.shape
    h = x.reshape(-1, D)

    dph, slots, gmap, geff = _router_call(h, router_weights, w_down)
    slots_km = slots.T.reshape(N * K)
    gmap_km = gmap.T.reshape(N * K)

    pad = _dispatch_call(dph, slots_km)
    pout = _experts_call(pad, experts_w12, experts_w3)
    gathered = _combine_call(pout, gmap_km)
    shared = _shared_call(h, gate_w, up_w, down_w)
    out = _final_call(shared, gathered, geff, w_up)
    return out.reshape(Bb, Ss, D)


# speedup vs baseline: 2.0183x; 1.0648x over previous
import functools

import jax
import jax.numpy as jnp
from jax import lax
from jax.experimental import pallas as pl
from jax.experimental.pallas import tpu as pltpu
from jax.experimental.pallas import tpu_sc as plsc

D_MODEL = 1024
D_FF = 4096
E = 64
K = 2
BOTTLE = 256
ROUTED_FF = 3072
N = 2048
ROUTE_SCALE = float(2.0 ** 0.5)
MOE_SCALE = 0.5
MAX_LOAD = (N * K // E // 8 + 6) * 8
NUM_SLOTS = E * MAX_LOAD
PAD_ROWS = NUM_SLOTS + MAX_LOAD

NW = 32
CHUNK = (N * K) // NW

_BF = jnp.bfloat16
_F32 = jnp.float32


def _dot(a, b, dims):
    return lax.dot_general(a.astype(_BF), b.astype(_BF), (dims, ((), ())),
                           preferred_element_type=_F32)


def _router_body(h_ref, rw_ref, wd_ref, dph_ref, slots_ref, gmap_ref, geff_ref,
                 hbf_ref):
    h = h_ref[...]
    hbf_ref[...] = h.astype(_BF)
    logits = _dot(h, rw_ref[...], ((1,), (1,)))
    scores = jax.nn.sigmoid(logits)
    col = lax.broadcasted_iota(jnp.int32, (N, E), 1)
    m1 = jnp.max(scores, axis=1, keepdims=True)
    i1 = jnp.min(jnp.where(scores == m1, col, E), axis=1, keepdims=True)
    l1 = jnp.sum(jnp.where(col == i1, logits, 0.0), axis=1, keepdims=True)
    scores2 = jnp.where(col == i1, -1.0, scores)
    m2 = jnp.max(scores2, axis=1, keepdims=True)
    i2 = jnp.min(jnp.where(scores2 == m2, col, E), axis=1, keepdims=True)
    l2 = jnp.sum(jnp.where(col == i2, logits, 0.0), axis=1, keepdims=True)
    g1 = jax.nn.sigmoid(l1 - l2)
    g2 = jax.nn.sigmoid(l2 - l1)

    oh1 = (col == i1).astype(_F32)
    oh2 = (col == i2).astype(_F32)
    oht = oh1 + oh2
    r = lax.broadcasted_iota(jnp.int32, (N, N), 0)
    c = lax.broadcasted_iota(jnp.int32, (N, N), 1)
    ltri = (c < r).astype(_F32)
    cnt = _dot(ltri, oht, ((1,), (0,)))
    p1 = jnp.sum(cnt * oh1, axis=1, keepdims=True).astype(jnp.int32)
    p2 = jnp.sum(cnt * oh2, axis=1, keepdims=True).astype(jnp.int32)

    keep1 = p1 < MAX_LOAD
    keep2 = p2 < MAX_LOAD
    slot1 = i1 * MAX_LOAD + p1
    slot2 = i2 * MAX_LOAD + p2
    slots_ref[...] = jnp.concatenate(
        [jnp.where(keep1, slot1, NUM_SLOTS), jnp.where(keep2, slot2, NUM_SLOTS)],
        axis=1)
    gmap_ref[...] = jnp.concatenate(
        [jnp.where(keep1, slot1, 0), jnp.where(keep2, slot2, 0)], axis=1)
    geff_ref[...] = jnp.concatenate(
        [jnp.where(keep1, g1, 0.0), jnp.where(keep2, g2, 0.0)], axis=1)

    dph_ref[...] = _dot(h, wd_ref[...], ((1,), (1,)))


def _router_call(h, router_weights, w_down):
    return pl.pallas_call(
        _router_body,
        out_shape=(
            jax.ShapeDtypeStruct((N, BOTTLE), _F32),
            jax.ShapeDtypeStruct((N, K), jnp.int32),
            jax.ShapeDtypeStruct((N, K), jnp.int32),
            jax.ShapeDtypeStruct((N, K), _F32),
            jax.ShapeDtypeStruct((N, D_MODEL), _BF),
        ),
    )(h, router_weights, w_down)


def _dispatch_call(dph, slots_km):
    mesh = plsc.VectorSubcoreMesh(core_axis_name="c", subcore_axis_name="s")

    @functools.partial(
        pl.kernel,
        out_type=jax.ShapeDtypeStruct((PAD_ROWS, BOTTLE), _F32),
        mesh=mesh,
        scratch_types=[
            pltpu.VMEM((CHUNK,), jnp.int32),
            pltpu.VMEM((CHUNK, BOTTLE), _F32),
            pltpu.SemaphoreType.DMA,
        ],
    )
    def disp(dph_hbm, slots_hbm, pad_hbm, idx_v, rows_v, sem):
        wid = lax.axis_index("s") * 2 + lax.axis_index("c")
        pltpu.sync_copy(slots_hbm.at[pl.ds(wid * CHUNK, CHUNK)], idx_v)
        tok0 = (wid % 16) * CHUNK
        pltpu.sync_copy(dph_hbm.at[pl.ds(tok0, CHUNK)], rows_v)
        pltpu.async_copy(rows_v, pad_hbm.at[idx_v], sem).wait()

    return disp(dph, slots_km)


FF_C = 256
FF_STEPS = D_FF // FF_C


def _experts_body(pad_ref, w12_ref, w3_ref, hbf_ref, gw_ref, uw_ref, dw_ref,
                  out_ref, sh_ref):
    e = pl.program_id(0)
    x = pad_ref[...]
    h12 = _dot(x, w12_ref[0], ((1,), (0,)))
    hg = h12[:, :ROUTED_FF]
    hu = h12[:, ROUTED_FF:]
    act = hg * jax.nn.sigmoid(hg) * hu
    out_ref[...] = _dot(act, w3_ref[0], ((1,), (0,)))

    @pl.when(e < FF_STEPS)
    def _():
        hbf = hbf_ref[...]
        g = _dot(hbf, gw_ref[...], ((1,), (1,)))
        u = _dot(hbf, uw_ref[...], ((1,), (1,)))
        a = g * jax.nn.sigmoid(g) * u
        part = _dot(a, dw_ref[...], ((1,), (1,)))

        @pl.when(e == 0)
        def _():
            sh_ref[...] = part

        @pl.when(e > 0)
        def _():
            sh_ref[...] += part


def _experts_call(pad, experts_w12, experts_w3, hbf, gate_w, up_w, down_w):
    clamp = lambda e: jnp.minimum(e, FF_STEPS - 1)
    return pl.pallas_call(
        _experts_body,
        grid=(E,),
        in_specs=[
            pl.BlockSpec((MAX_LOAD, BOTTLE), lambda e: (e, 0)),
            pl.BlockSpec((1, BOTTLE, 2 * ROUTED_FF), lambda e: (e, 0, 0)),
            pl.BlockSpec((1, ROUTED_FF, BOTTLE), lambda e: (e, 0, 0)),
            pl.BlockSpec((N, D_MODEL), lambda e: (0, 0)),
            pl.BlockSpec((FF_C, D_MODEL), lambda e: (clamp(e), 0)),
            pl.BlockSpec((FF_C, D_MODEL), lambda e: (clamp(e), 0)),
            pl.BlockSpec((D_MODEL, FF_C), lambda e: (0, clamp(e))),
        ],
        out_specs=(
            pl.BlockSpec((MAX_LOAD, BOTTLE), lambda e: (e, 0)),
            pl.BlockSpec((N, D_MODEL), lambda e: (0, 0)),
        ),
        out_shape=(
            jax.ShapeDtypeStruct((NUM_SLOTS, BOTTLE), _F32),
            jax.ShapeDtypeStruct((N, D_MODEL), _F32),
        ),
        compiler_params=pltpu.CompilerParams(
            dimension_semantics=("arbitrary",)),
    )(pad, experts_w12, experts_w3, hbf, gate_w, up_w, down_w)


def _combine_call(pout, gmap_km):
    mesh = plsc.VectorSubcoreMesh(core_axis_name="c", subcore_axis_name="s")

    @functools.partial(
        pl.kernel,
        out_type=jax.ShapeDtypeStruct((N * K, BOTTLE), _F32),
        mesh=mesh,
        scratch_types=[
            pltpu.VMEM((CHUNK,), jnp.int32),
            pltpu.VMEM((CHUNK, BOTTLE), _F32),
            pltpu.SemaphoreType.DMA,
        ],
    )
    def comb(pout_hbm, gmap_hbm, out_hbm, idx_v, rows_v, sem):
        wid = lax.axis_index("s") * 2 + lax.axis_index("c")
        pltpu.sync_copy(gmap_hbm.at[pl.ds(wid * CHUNK, CHUNK)], idx_v)
        pltpu.async_copy(pout_hbm.at[idx_v], rows_v, sem).wait()
        pltpu.sync_copy(rows_v, out_hbm.at[pl.ds(wid * CHUNK, CHUNK)])

    return comb(pout, gmap_km)


def _final_body(sh_ref, gath_ref, geff_ref, wu_ref, out_ref):
    g0 = geff_ref[:, 0:1]
    g1 = geff_ref[:, 1:2]
    routed_b = g0 * gath_ref[0:N, :] + g1 * gath_ref[N:2 * N, :]
    routed = _dot(routed_b, wu_ref[...], ((1,), (1,)))
    out_ref[...] = (sh_ref[...] + routed * ROUTE_SCALE) * MOE_SCALE


def _final_call(shared, gathered, geff, w_up):
    return pl.pallas_call(
        _final_body,
        out_shape=jax.ShapeDtypeStruct((N, D_MODEL), _F32),
    )(shared, gathered, geff, w_up)


def kernel(x, router_weights, gate_w, up_w, down_w, w_down, w_up,
           experts_w12, experts_w3):
    Bb, Ss, D = x.shape
    h = x.reshape(-1, D)

    dph, slots, gmap, geff, hbf = _router_call(h, router_weights, w_down)
    slots_km = slots.T.reshape(N * K)
    gmap_km = gmap.T.reshape(N * K)
    pad = _dispatch_call(dph, slots_km)
    pout, shared = _experts_call(pad, experts_w12, experts_w3, hbf,
                                 gate_w, up_w, down_w)
    gathered = _combine_call(pout, gmap_km)
    out = _final_call(shared, gathered, geff, w_up)
    return out.reshape(Bb, Ss, D)
